# baseline (speedup 1.0000x reference)
import functools

import jax
import jax.numpy as jnp
from jax import lax
from jax.experimental import pallas as pl
from jax.experimental.pallas import tpu as pltpu
from jax.experimental.pallas import tpu_sc as plsc


@functools.cache
def _make_lookup(Bt, F, D):
    info = plsc.get_sparse_core_info()
    NC, NS = info.num_cores, info.num_subcores
    NW = NC * NS
    KD = D // 8
    NBT = Bt // 128
    B = Bt * F
    n_per_w = B // NW
    nb_w = Bt // NW
    SB = 32
    R = SB * F
    nsb = nb_w // SB
    assert nsb * R == n_per_w and NBT % NW == 0 and 128 % SB == 0

    mesh = plsc.VectorSubcoreMesh(core_axis_name="c", subcore_axis_name="s")

    @functools.partial(
        pl.kernel,
        mesh=mesh,
        out_type=jax.ShapeDtypeStruct((F, KD, NBT, 8, 128), jnp.float32),
        scratch_types=[
            pltpu.VMEM((n_per_w,), jnp.int32),
            pltpu.VMEM((R, D), jnp.float32),
            pltpu.VMEM((R, D), jnp.float32),
            pltpu.VMEM((F, KD, 8, SB), jnp.float32),
            pltpu.VMEM((F, KD, 8, SB), jnp.float32),
            pltpu.SemaphoreType.DMA,
            pltpu.SemaphoreType.DMA,
            pltpu.SemaphoreType.DMA,
            pltpu.SemaphoreType.DMA,
        ],
        compiler_params=pltpu.CompilerParams(
            use_tc_tiling_on_sc=False, needs_layout_passes=False
        ),
    )
    def k(idx_hbm, table_hbm, q_hbm, idx_v, rows_a, rows_b, tb_a, tb_b,
          g_a, g_b, o_a, o_b):
        wid = lax.axis_index("s") * NC + lax.axis_index("c")
        base = wid * n_per_w
        bt_base = wid * (NBT // NW)
        pltpu.sync_copy(idx_hbm.at[pl.ds(base, n_per_w)], idx_v)

        rows = (rows_a, rows_b)
        tb = (tb_a, tb_b)
        gsem = (g_a, g_b)
        osem = (o_a, o_b)

        def transpose_block(rows_ref, tb_ref):
            @plsc.parallel_loop(0, F, 1, unroll=4)
            def body(f):
                lane = jax.lax.iota(jnp.int32, 16)
                for bqg in range(SB // 16):
                    row_vec = (bqg * 16 + lane) * F + f
                    for kk in range(KD):
                        for ss in range(8):
                            d = kk * 8 + ss
                            col = jnp.full((16,), d, dtype=jnp.int32)
                            g = plsc.load_gather(rows_ref, [row_vec, col])
                            tb_ref[f, kk, ss, pl.ds(bqg * 16, 16)] = g

        def pair_body(g, carry):
            j0 = 2 * g
            g0 = pltpu.async_copy(
                table_hbm.at[idx_v.at[pl.ds(j0 * R, R)]], rows[0], gsem[0]
            )
            g1 = pltpu.async_copy(
                table_hbm.at[idx_v.at[pl.ds(j0 * R + R, R)]], rows[1], gsem[1]
            )
            g0.wait()
            transpose_block(rows[0], tb[0])
            bt0 = bt_base + (j0 * SB) // 128
            bl0 = (j0 * SB) % 128
            s0 = pltpu.async_copy(
                tb[0], q_hbm.at[:, :, bt0, :, pl.ds(bl0, SB)], osem[0]
            )
            g1.wait()
            transpose_block(rows[1], tb[1])
            bt1 = bt_base + ((j0 + 1) * SB) // 128
            bl1 = ((j0 + 1) * SB) % 128
            s1 = pltpu.async_copy(
                tb[1], q_hbm.at[:, :, bt1, :, pl.ds(bl1, SB)], osem[1]
            )
            s0.wait()
            s1.wait()
            return carry

        lax.fori_loop(0, nsb // 2, pair_body, 0)

    return k


@jax.jit
def kernel(x, table):
    Bt, F = x.shape
    V, D = table.shape
    flat_idx = x.reshape(Bt * F)
    q = _make_lookup(Bt, F, D)(flat_idx, table)
    return q.transpose(2, 4, 0, 1, 3).reshape(Bt, F, D)


# speedup vs baseline: 1.2808x; 1.2808x over previous
import functools

import jax
import jax.numpy as jnp
import numpy as np
from jax import lax
from jax.experimental import pallas as pl
from jax.experimental.pallas import tpu as pltpu
from jax.experimental.pallas import tpu_sc as plsc


@functools.cache
def _make_lookup(Bt, F, D):
    info = plsc.get_sparse_core_info()
    NC, NS = info.num_cores, info.num_subcores
    NW = NC * NS
    KD = D // 8
    NBT = Bt // 128
    B = Bt * F
    n_per_w = B // NW
    nb_w = Bt // NW
    SB = 32
    R = SB * F
    nsb = nb_w // SB
    assert nsb * R == n_per_w and NBT % NW == 0 and 128 % SB == 0

    mesh = plsc.VectorSubcoreMesh(core_axis_name="c", subcore_axis_name="s")

    @functools.partial(
        pl.kernel,
        mesh=mesh,
        out_type=jax.ShapeDtypeStruct((F, KD, NBT, 8, 128), jnp.float32),
        scratch_types=[
            pltpu.VMEM((n_per_w,), jnp.int32),
            pltpu.VMEM((R, D), jnp.float32),
            pltpu.VMEM((R, D), jnp.float32),
            pltpu.VMEM((F, KD, 8, SB), jnp.float32),
            pltpu.VMEM((F, KD, 8, SB), jnp.float32),
            pltpu.SemaphoreType.DMA,
            pltpu.SemaphoreType.DMA,
            pltpu.SemaphoreType.DMA,
            pltpu.SemaphoreType.DMA,
        ],
        compiler_params=pltpu.CompilerParams(
            use_tc_tiling_on_sc=False, needs_layout_passes=False
        ),
    )
    def k(idx_hbm, table_hbm, q_hbm, idx_v, rows_a, rows_b, tb_a, tb_b,
          g_a, g_b, o_a, o_b):
        wid = lax.axis_index("s") * NC + lax.axis_index("c")
        base = wid * n_per_w
        bt_base = wid * (NBT // NW)
        pltpu.sync_copy(idx_hbm.at[pl.ds(base, n_per_w)], idx_v)

        rows = (rows_a, rows_b)
        tb = (tb_a, tb_b)
        gsem = (g_a, g_b)
        osem = (o_a, o_b)

        ar16 = np.arange(16)

        def transpose_block(rows_ref, tb_ref):
            @plsc.parallel_loop(0, F, 1, unroll=2)
            def body(f):
                lane = jax.lax.iota(jnp.int32, 16)
                tbf = tb_ref.at[f]
                for bqg in range(SB // 16):
                    row_vec = (bqg * 16 + lane) * F + f
                    bq_c = bqg * 16 + lane
                    for h in range(D // 16):
                        for c in range(16):
                            col = 16 * h + ((lane + c) % 16)
                            kk_c = col // 8
                            ss_c = col % 8
                            v = plsc.load_gather(rows_ref, [row_vec, col])
                            plsc.store_scatter(tbf, [kk_c, ss_c, bq_c], v)

        def pair_body(g, carry):
            j0 = 2 * g
            g0 = pltpu.async_copy(
                table_hbm.at[idx_v.at[pl.ds(j0 * R, R)]], rows[0], gsem[0]
            )
            g1 = pltpu.async_copy(
                table_hbm.at[idx_v.at[pl.ds(j0 * R + R, R)]], rows[1], gsem[1]
            )
            g0.wait()
            transpose_block(rows[0], tb[0])
            bt0 = bt_base + (j0 * SB) // 128
            bl0 = (j0 * SB) % 128
            s0 = pltpu.async_copy(
                tb[0], q_hbm.at[:, :, bt0, :, pl.ds(bl0, SB)], osem[0]
            )
            g1.wait()
            transpose_block(rows[1], tb[1])
            bt1 = bt_base + ((j0 + 1) * SB) // 128
            bl1 = ((j0 + 1) * SB) % 128
            s1 = pltpu.async_copy(
                tb[1], q_hbm.at[:, :, bt1, :, pl.ds(bl1, SB)], osem[1]
            )
            s0.wait()
            s1.wait()
            return carry

        lax.fori_loop(0, nsb // 2, pair_body, 0)

    return k


@jax.jit
def kernel(x, table):
    Bt, F = x.shape
    V, D = table.shape
    flat_idx = x.reshape(Bt * F)
    q = _make_lookup(Bt, F, D)(flat_idx, table)
    return q.transpose(2, 4, 0, 1, 3).reshape(Bt, F, D)
